# Initial kernel scaffold; baseline (speedup 1.0000x reference)
import jax
import jax.numpy as jnp
from jax.experimental import pallas as pl


def kernel(logits, labels):
    raise NotImplementedError("write your pallas kernel here")


# speedup vs baseline: 22.1505x; 22.1505x over previous
import functools

import jax
import jax.numpy as jnp
from jax import lax
from jax.experimental import pallas as pl
from jax.experimental.pallas import tpu as pltpu
from jax.experimental.pallas import tpu_sc as plsc

K = 8192
NC, NS = 2, 16
NW = NC * NS
CHUNKS_PER_CLASS = 4
SUB = 16384


def _softmax_body(x_ref, o_ref):
    x = x_ref[...]
    m = jnp.max(x, axis=1, keepdims=True)
    e = jnp.exp(x - m)
    p = e / jnp.sum(e, axis=1, keepdims=True)
    o_ref[...] = jnp.reshape(p, o_ref.shape)


def _softmax_classmajor(logits):
    B, C, H, W = logits.shape
    return pl.pallas_call(
        _softmax_body,
        grid=(B, H // 8),
        in_specs=[pl.BlockSpec((1, C, 8, W), lambda b, i: (b, 0, i, 0))],
        out_specs=pl.BlockSpec((C, 1, 8, W), lambda b, i: (0, b, i, 0)),
        out_shape=jax.ShapeDtypeStruct((C, B, H, W), jnp.float32),
    )(logits)


def _sc_hist_body(nt, n_per_class, c_probs, probs_hbm, labels_hbm, out_hbm,
                  table, pbuf, lbuf):
    chunk_elems = n_per_class // CHUNKS_PER_CLASS
    nsub = chunk_elems // SUB
    wid = lax.axis_index("s") * NC + lax.axis_index("c")
    ntasks_per_worker = (nt + NW - 1) // NW

    for ti in range(ntasks_per_worker):
        t = wid + NW * ti

        @pl.when(t < nt)
        def _process():
            c = t // CHUNKS_PER_CLASS
            chunk = t % CHUNKS_PER_CLASS

            def zero_body(j, carry):
                table[pl.ds(j * 16, 16)] = jnp.zeros((16,), jnp.float32)
                return carry

            lax.fori_loop(0, (4 * K) // 16, zero_body, 0)

            def sub_body(s, carry):
                start = chunk * chunk_elems + s * SUB
                pltpu.sync_copy(probs_hbm.at[c, pl.ds(start, SUB)], pbuf)
                pltpu.sync_copy(labels_hbm.at[pl.ds(start, SUB)], lbuf)

                def elem_body(i, icarry):
                    p = pbuf[pl.ds(i * 16, 16)]
                    l = lbuf[pl.ds(i * 16, 16)]
                    flag = l == c
                    e = jnp.where(flag, 1.0 - p, p)
                    kb = jnp.minimum((e * float(K)).astype(jnp.int32), K - 1)
                    base = jnp.where(flag, 0, 2 * K) + kb
                    plsc.addupdate_scatter(table, [base],
                                           jnp.ones((16,), jnp.float32))
                    plsc.addupdate_scatter(table, [base + K], e)
                    return icarry

                lax.fori_loop(0, SUB // 16, elem_body, 0)
                return carry

            lax.fori_loop(0, nsub, sub_body, 0)
            pltpu.sync_copy(table, out_hbm.at[t])


def _sc_hist(probs_cm, labels_flat):
    c_probs, n = probs_cm.shape
    nt = c_probs * CHUNKS_PER_CLASS
    mesh = plsc.VectorSubcoreMesh(core_axis_name="c", subcore_axis_name="s")
    body = functools.partial(_sc_hist_body, nt, n, c_probs)
    return pl.kernel(
        body,
        mesh=mesh,
        compiler_params=pltpu.CompilerParams(needs_layout_passes=False),
        out_type=jax.ShapeDtypeStruct((nt, 4 * K), jnp.float32),
        scratch_types=[
            pltpu.VMEM((4 * K,), jnp.float32),
            pltpu.VMEM((SUB,), jnp.float32),
            pltpu.VMEM((SUB,), jnp.int32),
        ],
    )(probs_cm, labels_flat)


def _finalize_body(t_ref, o_ref):
    nt, _, rr, ll = t_ref.shape
    c_cls = nt // CHUNKS_PER_CLASS
    w = 1.0 / float(K)

    x = t_ref[...]
    x = jnp.reshape(x, (c_cls, CHUNKS_PER_CLASS, 4, rr, ll))
    x = jnp.sum(x, axis=1)
    cnt_f = x[:, 0]
    m_f = x[:, 1]
    cnt_b = x[:, 2]
    m_b = x[:, 3]

    ii = lax.broadcasted_iota(jnp.int32, (ll, ll), 0)
    jj = lax.broadcasted_iota(jnp.int32, (ll, ll), 1)
    lane_tri = (ii <= jj).astype(jnp.float32)
    i2 = lax.broadcasted_iota(jnp.int32, (rr, rr), 0)
    j2 = lax.broadcasted_iota(jnp.int32, (rr, rr), 1)
    row_tri = (i2 < j2).astype(jnp.float32)

    def incl_cumsum(y):
        flat = jnp.reshape(y, (c_cls * rr, ll))
        rowp = jnp.dot(flat, lane_tri, preferred_element_type=jnp.float32)
        rowp = jnp.reshape(rowp, (c_cls, rr, ll))
        sums = jnp.sum(y, axis=-1)
        blockp = jnp.dot(sums, row_tri, preferred_element_type=jnp.float32)
        return rowp + blockp[:, :, None]

    incl_f = incl_cumsum(cnt_f)
    incl_b = incl_cumsum(cnt_b)

    g = jnp.sum(jnp.sum(cnt_f, axis=-1), axis=-1)
    tot_b = jnp.sum(jnp.sum(cnt_b, axis=-1), axis=-1)
    g3 = g[:, None, None]
    f_above = g3 - incl_f
    b_above = tot_b[:, None, None] - incl_b

    kr = lax.broadcasted_iota(jnp.int32, (c_cls, rr, ll), 1)
    kl = lax.broadcasted_iota(jnp.int32, (c_cls, rr, ll), 2)
    u = (kr * ll + kl).astype(jnp.float32) * w

    uu = jnp.maximum(g3 + b_above, 1.0)
    phi = g3 - f_above
    sf = m_f - cnt_f * u
    sb = m_b - cnt_b * u
    inv_u = 1.0 / uu
    t_term = (phi * w - sf) * inv_u
    t_term = t_term - phi * sb * inv_u * inv_u
    t_term = t_term + (cnt_f * cnt_b + phi * cnt_b * cnt_b * inv_u) \
        * (w / 3.0) * inv_u * inv_u

    t_sum = jnp.sum(jnp.sum(t_term, axis=-1), axis=-1)
    loss_c = 1.0 - t_sum
    present = (g > 0.5).astype(jnp.float32)
    num = jnp.sum(loss_c * present)
    den = jnp.maximum(jnp.sum(present), 1.0)
    o_ref[...] = jnp.reshape(num / den, (1, 1))


def _finalize(tables):
    out = pl.pallas_call(
        _finalize_body,
        out_shape=jax.ShapeDtypeStruct((1, 1), jnp.float32),
    )(tables)
    return out


def kernel(logits, labels):
    B, C, H, W = logits.shape
    probs_t = _softmax_classmajor(logits)
    probs_cm = probs_t.reshape(C, B * H * W)
    labels_flat = labels.reshape(-1).astype(jnp.int32)
    tables = _sc_hist(probs_cm, labels_flat)
    tables4 = tables.reshape(C * CHUNKS_PER_CLASS, 4, K // 128, 128)
    out = _finalize(tables4)
    return out.reshape(())
